# baseline (speedup 1.0000x reference)
import functools

import jax
import jax.numpy as jnp
from jax import lax
from jax.experimental import pallas as pl
from jax.experimental.pallas import tpu as pltpu
from jax.experimental.pallas import tpu_sc as plsc

_NC = 1
_NS = 16
_NW = _NC * _NS
_L = 16

_N = 100000
_NSPEC = 119
_CPW = -(-_N // (_NW * _L)) * _L


@functools.partial(
    pl.kernel,
    mesh=plsc.VectorSubcoreMesh(
        core_axis_name="c", subcore_axis_name="s", num_cores=1
    ),
    compiler_params=pltpu.CompilerParams(needs_layout_passes=False),
    out_type=jax.ShapeDtypeStruct((_N,), jnp.float32),
    scratch_types=[
        pltpu.VMEM((_NSPEC,), jnp.float32),
        pltpu.VMEM((_CPW,), jnp.int32),
        pltpu.VMEM((_CPW,), jnp.float32),
        pltpu.SemaphoreType.DMA,
    ],
)
def _rescale(e_hbm, s_hbm, v_hbm, out_hbm, table_v, idx_v, e_v, sem):
    wid = lax.axis_index("s") * _NC + lax.axis_index("c")
    base = jnp.minimum(wid * _CPW, _N - _CPW)
    cp_t = pltpu.async_copy(v_hbm, table_v, sem)
    cp_s = pltpu.async_copy(s_hbm.at[pl.ds(base, _CPW)], idx_v, sem)
    cp_e = pltpu.async_copy(e_hbm.at[pl.ds(base, _CPW)], e_v, sem)
    cp_t.wait()
    cp_s.wait()
    cp_e.wait()

    @plsc.parallel_loop(0, _CPW, step=_L, unroll=8)
    def body(i):
        sl = pl.ds(i, _L)
        g = plsc.load_gather(table_v, [idx_v[sl]])
        e_v[sl] = e_v[sl] + g

    pltpu.sync_copy(e_v, out_hbm.at[pl.ds(base, _CPW)])


def kernel(energies, node_species, values):
    return _rescale(energies, node_species, values)


# speedup vs baseline: 1.0060x; 1.0060x over previous
import functools

import jax
import jax.numpy as jnp
from jax import lax
from jax.experimental import pallas as pl
from jax.experimental.pallas import tpu as pltpu
from jax.experimental.pallas import tpu_sc as plsc

_NC = 1
_NS = 16
_NW = _NC * _NS
_L = 16

_N = 100000
_NSPEC = 119
_CPW = -(-_N // (_NW * _L)) * _L


@functools.partial(
    pl.kernel,
    mesh=plsc.VectorSubcoreMesh(
        core_axis_name="c", subcore_axis_name="s", num_cores=1
    ),
    compiler_params=pltpu.CompilerParams(needs_layout_passes=False),
    out_type=jax.ShapeDtypeStruct((_N,), jnp.float32),
    scratch_types=[
        pltpu.VMEM((_NSPEC,), jnp.float32),
        pltpu.VMEM((_CPW,), jnp.int32),
        pltpu.VMEM((_CPW,), jnp.float32),
        pltpu.SemaphoreType.DMA,
        pltpu.SemaphoreType.DMA,
        pltpu.SemaphoreType.DMA,
    ],
)
def _rescale(e_hbm, s_hbm, v_hbm, out_hbm, table_v, idx_v, e_v, sem0, sem1, sem_o):
    wid = lax.axis_index("s") * _NC + lax.axis_index("c")
    base = jnp.minimum(wid * _CPW, _N - _CPW)
    _H = _CPW // 2
    cp_t = pltpu.async_copy(v_hbm, table_v, sem0)
    cp_s0 = pltpu.async_copy(s_hbm.at[pl.ds(base, _H)], idx_v.at[pl.ds(0, _H)], sem0)
    cp_e0 = pltpu.async_copy(e_hbm.at[pl.ds(base, _H)], e_v.at[pl.ds(0, _H)], sem0)
    cp_s1 = pltpu.async_copy(s_hbm.at[pl.ds(base + _H, _H)], idx_v.at[pl.ds(_H, _H)], sem1)
    cp_e1 = pltpu.async_copy(e_hbm.at[pl.ds(base + _H, _H)], e_v.at[pl.ds(_H, _H)], sem1)
    cp_t.wait()
    cp_s0.wait()
    cp_e0.wait()

    @plsc.parallel_loop(0, _H, step=_L, unroll=8)
    def body0(i):
        sl = pl.ds(i, _L)
        g = plsc.load_gather(table_v, [idx_v[sl]])
        e_v[sl] = e_v[sl] + g

    cp_o0 = pltpu.async_copy(e_v.at[pl.ds(0, _H)], out_hbm.at[pl.ds(base, _H)], sem_o)
    cp_s1.wait()
    cp_e1.wait()

    @plsc.parallel_loop(_H, _CPW, step=_L, unroll=8)
    def body1(i):
        sl = pl.ds(i, _L)
        g = plsc.load_gather(table_v, [idx_v[sl]])
        e_v[sl] = e_v[sl] + g

    cp_o1 = pltpu.async_copy(e_v.at[pl.ds(_H, _H)], out_hbm.at[pl.ds(base + _H, _H)], sem_o)
    cp_o0.wait()
    cp_o1.wait()


def kernel(energies, node_species, values):
    return _rescale(energies, node_species, values)
